# baseline (device time: 94562 ns/iter reference)
import jax
import jax.numpy as jnp
from jax import lax
from jax.experimental import pallas as pl
from jax.experimental.pallas import tpu as pltpu

N_DEV = 4


def kernel(x, Win0, Wout0, Win1, Wout1, Win2, Wout2):
    m_per, d = x.shape
    M = N_DEV * m_per

    def body(x_ref, win0_ref, wout0_ref, win1_ref, wout1_ref, win2_ref,
             wout2_ref, out_ref, xfull_ref, p_ref, xcur_ref,
             ag_comm, rs_comm, ag_send_sems, ag_recv_sems,
             rs_send_sems, rs_recv_sems):
        j = lax.axis_index("i")
        left = lax.rem(j + N_DEV - 1, N_DEV)
        right = lax.rem(j + 1, N_DEV)

        barrier_sem = pltpu.get_barrier_semaphore()
        for nbr in (left, right):
            pl.semaphore_signal(barrier_sem, inc=1, device_id=(nbr,),
                                device_id_type=pl.DeviceIdType.MESH)
        pl.semaphore_wait(barrier_sem, 2)

        xcur_ref[...] = x_ref[...]

        layers = [(win0_ref, wout0_ref), (win1_ref, wout1_ref),
                  (win2_ref, wout2_ref)]
        for l, (win_ref, wout_ref) in enumerate(layers):
            xfull_ref[pl.ds(j * m_per, m_per), :] = xcur_ref[...]
            ag_comm[0] = xcur_ref[...]
            for h in range(N_DEV - 1):
                s_slot, r_slot = h % 2, (h + 1) % 2
                rdma = pltpu.make_async_remote_copy(
                    src_ref=ag_comm.at[s_slot],
                    dst_ref=ag_comm.at[r_slot],
                    send_sem=ag_send_sems.at[s_slot],
                    recv_sem=ag_recv_sems.at[r_slot],
                    device_id=(right,),
                    device_id_type=pl.DeviceIdType.MESH,
                )
                rdma.start()
                rdma.wait()
                origin = lax.rem(j + N_DEV - h - 1, N_DEV)
                xfull_ref[pl.ds(origin * m_per, m_per), :] = ag_comm[r_slot]

            hact = jnp.maximum(
                jnp.dot(xfull_ref[...], win_ref[...],
                        preferred_element_type=jnp.float32), 0.0)
            p_ref[...] = jnp.dot(hact, wout_ref[...],
                                 preferred_element_type=jnp.float32)

            first = lax.rem(j + N_DEV - 1, N_DEV)
            rs_comm[0] = p_ref[pl.ds(first * m_per, m_per), :]
            for s in range(N_DEV - 1):
                s_slot, r_slot = s % 2, (s + 1) % 2
                rdma = pltpu.make_async_remote_copy(
                    src_ref=rs_comm.at[s_slot],
                    dst_ref=rs_comm.at[r_slot],
                    send_sem=rs_send_sems.at[s_slot],
                    recv_sem=rs_recv_sems.at[r_slot],
                    device_id=(right,),
                    device_id_type=pl.DeviceIdType.MESH,
                )
                rdma.start()
                rdma.wait()
                c = lax.rem(j + 2 * N_DEV - s - 2, N_DEV)
                acc = rs_comm[r_slot] + p_ref[pl.ds(c * m_per, m_per), :]
                if s < N_DEV - 2:
                    rs_comm[r_slot] = acc
                elif l < len(layers) - 1:
                    xcur_ref[...] = acc
                else:
                    out_ref[...] = acc

    return pl.pallas_call(
        body,
        out_shape=jax.ShapeDtypeStruct((m_per, d), jnp.float32),
        in_specs=[pl.BlockSpec(memory_space=pltpu.VMEM)] * 7,
        out_specs=pl.BlockSpec(memory_space=pltpu.VMEM),
        scratch_shapes=[
            pltpu.VMEM((M, d), jnp.float32),
            pltpu.VMEM((M, d), jnp.float32),
            pltpu.VMEM((m_per, d), jnp.float32),
            pltpu.VMEM((2, m_per, d), jnp.float32),
            pltpu.VMEM((2, m_per, d), jnp.float32),
            pltpu.SemaphoreType.DMA((2,)),
            pltpu.SemaphoreType.DMA((2,)),
            pltpu.SemaphoreType.DMA((2,)),
            pltpu.SemaphoreType.DMA((2,)),
        ],
        compiler_params=pltpu.CompilerParams(collective_id=0),
    )(x, Win0, Wout0, Win1, Wout1, Win2, Wout2)


# device time: 66287 ns/iter; 1.4266x vs baseline; 1.4266x over previous
import jax
import jax.numpy as jnp
from jax import lax
from jax.experimental import pallas as pl
from jax.experimental.pallas import tpu as pltpu

N_DEV = 4


def kernel(x, Win0, Wout0, Win1, Wout1, Win2, Wout2):
    m_per, d = x.shape

    def body(x_ref, win0_ref, wout0_ref, win1_ref, wout1_ref, win2_ref,
             wout2_ref, out_ref,
             xcur, ag1L, ag1R, ag2, rsA, rb_fL, rb_fR,
             pj, pL, pR, pD, stage, ssem, rsem):
        j = lax.axis_index("i")
        left = lax.rem(j + N_DEV - 1, N_DEV)
        right = lax.rem(j + 1, N_DEV)

        barrier_sem = pltpu.get_barrier_semaphore()
        for nbr in (left, right):
            pl.semaphore_signal(barrier_sem, inc=1, device_id=(nbr,),
                                device_id_type=pl.DeviceIdType.MESH)
        pl.semaphore_wait(barrier_sem, 2)

        def mlp(src_ref, win_ref, wout_ref):
            h = jnp.maximum(
                jnp.dot(src_ref[...], win_ref[...],
                        preferred_element_type=jnp.float32), 0.0)
            return jnp.dot(h, wout_ref[...],
                           preferred_element_type=jnp.float32)

        def copy(src, dst, s, r, dev):
            return pltpu.make_async_remote_copy(
                src_ref=src, dst_ref=dst, send_sem=ssem.at[s],
                recv_sem=rsem.at[r], device_id=(dev,),
                device_id_type=pl.DeviceIdType.MESH)

        xcur[...] = x_ref[...]

        layers = [(win0_ref, wout0_ref), (win1_ref, wout1_ref),
                  (win2_ref, wout2_ref)]
        for l, (win_ref, wout_ref) in enumerate(layers):
            r1r = copy(xcur, ag1L, 0, 0, right)
            r1l = copy(xcur, ag1R, 1, 1, left)
            r1r.start()
            r1l.start()
            pj[...] = mlp(xcur, win_ref, wout_ref)
            r1r.wait()
            r1l.wait()

            r2 = copy(ag1L, ag2, 2, 2, right)
            r2.start()
            pL[...] = mlp(ag1L, win_ref, wout_ref)
            pR[...] = mlp(ag1R, win_ref, wout_ref)
            r2.wait()
            pD[...] = mlp(ag2, win_ref, wout_ref)

            ra = copy(pD, rsA, 3, 3, left)
            ra.start()
            ra.wait()

            stage[...] = pL[...] + rsA[...]
            rbl = copy(stage, rb_fR, 4, 4, left)
            rbr = copy(pR, rb_fL, 5, 5, right)
            rbl.start()
            rbr.start()
            rbl.wait()
            rbr.wait()

            res = pj[...] + rb_fL[...] + rb_fR[...]
            if l < len(layers) - 1:
                xcur[...] = res
            else:
                out_ref[...] = res

    buf = lambda: pltpu.VMEM((m_per, d), jnp.float32)
    return pl.pallas_call(
        body,
        out_shape=jax.ShapeDtypeStruct((m_per, d), jnp.float32),
        in_specs=[pl.BlockSpec(memory_space=pltpu.VMEM)] * 7,
        out_specs=pl.BlockSpec(memory_space=pltpu.VMEM),
        scratch_shapes=[
            buf(),
            buf(),
            buf(),
            buf(),
            buf(),
            buf(),
            buf(),
            buf(),
            buf(),
            buf(),
            buf(),
            buf(),
            pltpu.SemaphoreType.DMA((6,)),
            pltpu.SemaphoreType.DMA((6,)),
        ],
        compiler_params=pltpu.CompilerParams(collective_id=0),
    )(x, Win0, Wout0, Win1, Wout1, Win2, Wout2)
